# baseline (speedup 1.0000x reference)
import functools

import jax
import jax.numpy as jnp
from jax import lax
from jax.experimental import pallas as pl
from jax.experimental.pallas import tpu as pltpu
from jax.experimental.pallas import tpu_sc as plsc

NC = 2
NS = 16
NW = NC * NS
CPW = 4
CHUNK = 128
BLKI = 32
L = 16


def _sc_agg_body(npad, nblk, hcols, idx3, zer, out,
                 ib0, ib1, h_t, acc_t, si0, si1):
    ib = (ib0, ib1)
    sem_i = (si0, si1)
    c = lax.axis_index("c")
    s = lax.axis_index("s")
    wid = s * NC + c

    pltpu.sync_copy(hcols.at[wid], h_t)
    pltpu.sync_copy(zer, acc_t)
    pltpu.async_copy(idx3.at[pl.ds(0, BLKI)], ib[0], sem_i[0])
    pltpu.async_copy(idx3.at[pl.ds(BLKI, BLKI)], ib[1], sem_i[1])

    def do_block(ibt):
        def chunk(j, carry):
            for k in range(CHUNK // L):
                src = ibt[j, 0, pl.ds(k * L, L)] * CPW
                dst = ibt[j, 1, pl.ds(k * L, L)] * CPW
                for col in range(CPW):
                    v = plsc.load_gather(h_t, [src + col])
                    plsc.addupdate_scatter(acc_t, [dst + col], v)
            return carry
        lax.fori_loop(0, BLKI, chunk, 0)

    def pair(g, carry):
        for b in range(2):
            t = g * 2 + b
            pltpu.make_async_copy(
                idx3.at[pl.ds(t * BLKI, BLKI)], ib[b], sem_i[b]).wait()
            do_block(ib[b])

            @pl.when(t + 2 < nblk)
            def _():
                pltpu.async_copy(idx3.at[pl.ds((t + 2) * BLKI, BLKI)],
                                 ib[b], sem_i[b])
        return carry

    lax.fori_loop(0, nblk // 2, pair, 0)

    pltpu.sync_copy(acc_t, out.at[wid])


def _sc_aggregate(hcols, idx3, zer, npad, nblk):
    body = functools.partial(_sc_agg_body, npad, nblk)
    return pl.kernel(
        body,
        out_type=jax.ShapeDtypeStruct((NW, npad * CPW), jnp.float32),
        mesh=plsc.VectorSubcoreMesh(core_axis_name="c", subcore_axis_name="s"),
        compiler_params=pltpu.CompilerParams(needs_layout_passes=False),
        scratch_types=[
            pltpu.VMEM((BLKI, 2, CHUNK), jnp.int32),
            pltpu.VMEM((BLKI, 2, CHUNK), jnp.int32),
            pltpu.VMEM((hcols.shape[1],), jnp.float32),
            pltpu.VMEM((npad * CPW,), jnp.float32),
            pltpu.SemaphoreType.DMA,
            pltpu.SemaphoreType.DMA,
        ],
    )(hcols, idx3, zer)


def _tc_layer_body(n, aggp, h, scale, W1, b1, W2, b2, hout, sin, sout):
    agg = aggp[:n, :]
    hv = h[...]
    u = agg + scale[0, 0] * hv
    a1 = jnp.maximum(
        jnp.dot(u, W1[...], preferred_element_type=jnp.float32) + b1[...], 0.0)
    u2 = jnp.dot(a1, W2[...], preferred_element_type=jnp.float32) + b2[...]
    m = jnp.mean(u2, axis=0, keepdims=True)
    var = jnp.mean(u2 * u2, axis=0, keepdims=True) - m * m
    hn = jnp.maximum((u2 - m) * lax.rsqrt(var + 1e-5), 0.0)
    hout[...] = hn
    sin[...] = jnp.sum(hv, axis=0, keepdims=True)
    sout[...] = jnp.sum(hn, axis=0, keepdims=True)


def _tc_layer(aggp, h, scale, W1, b1, W2, b2):
    n, d = h.shape
    hdim = W1.shape[1]
    return pl.pallas_call(
        functools.partial(_tc_layer_body, n),
        out_shape=[
            jax.ShapeDtypeStruct((n, hdim), jnp.float32),
            jax.ShapeDtypeStruct((1, d), jnp.float32),
            jax.ShapeDtypeStruct((1, hdim), jnp.float32),
        ],
        in_specs=[
            pl.BlockSpec(memory_space=pltpu.VMEM),
            pl.BlockSpec(memory_space=pltpu.VMEM),
            pl.BlockSpec(memory_space=pltpu.SMEM),
            pl.BlockSpec(memory_space=pltpu.VMEM),
            pl.BlockSpec(memory_space=pltpu.VMEM),
            pl.BlockSpec(memory_space=pltpu.VMEM),
            pl.BlockSpec(memory_space=pltpu.VMEM),
        ],
        out_specs=[
            pl.BlockSpec(memory_space=pltpu.VMEM),
            pl.BlockSpec(memory_space=pltpu.VMEM),
            pl.BlockSpec(memory_space=pltpu.VMEM),
        ],
    )(aggp, h, scale, W1, b1, W2, b2)


def _score_body(sx, s1, s2, pw, pb, out):
    r = (jnp.dot(sx[...], pw[0], preferred_element_type=jnp.float32)
         + jnp.dot(s1[...], pw[1], preferred_element_type=jnp.float32)
         + jnp.dot(s2[...], pw[2], preferred_element_type=jnp.float32)
         + pb[...])
    out[...] = r


def _score(sx, s1, s2, pw, pb):
    d = pw.shape[2]
    return pl.pallas_call(
        _score_body,
        out_shape=jax.ShapeDtypeStruct((1, d), jnp.float32),
        in_specs=[pl.BlockSpec(memory_space=pltpu.VMEM)] * 5,
        out_specs=pl.BlockSpec(memory_space=pltpu.VMEM),
    )(sx, s1, s2, pw, pb)


def _colsplit(h, n):
    return h.reshape(n, NW, CPW).swapaxes(0, 1).reshape(NW, n * CPW)


def kernel(x, edge_index, eps, W1_0, b1_0, W2_0, b2_0, W1_1, b1_1, W2_1, b2_1,
           pW0, pb0, pW1, pb1, pW2, pb2):
    n, d = x.shape
    e = edge_index.shape[1]
    o = pW0.shape[1]

    rows = -(-e // CHUNK)
    q = 2 * BLKI
    rows_pad = -(-rows // q) * q
    nblk = rows_pad // BLKI
    epad = rows_pad * CHUNK
    npad = n + 8

    srcr = jnp.concatenate(
        [edge_index[0], jnp.zeros((epad - e,), jnp.int32)]).reshape(rows_pad, CHUNK)
    dstr = jnp.concatenate(
        [edge_index[1], jnp.full((epad - e,), n, jnp.int32)]).reshape(rows_pad, CHUNK)
    idx3 = jnp.stack([srcr, dstr], axis=1)
    zer = jnp.zeros((npad * CPW,), jnp.float32)

    scale0 = (1.0 + eps[0]).reshape(1, 1)
    scale1 = (1.0 + eps[1]).reshape(1, 1)
    b1_0r, b2_0r = b1_0.reshape(1, -1), b2_0.reshape(1, -1)
    b1_1r, b2_1r = b1_1.reshape(1, -1), b2_1.reshape(1, -1)

    def agg_full(h):
        a = _sc_aggregate(_colsplit(h, n), idx3, zer, npad, nblk)
        return a.reshape(NW, npad, CPW).swapaxes(0, 1).reshape(
            npad, d)[:n]

    agg0 = agg_full(x)
    h1, sx, s1 = _tc_layer(agg0, x, scale0, W1_0, b1_0r, W2_0, b2_0r)
    agg1 = agg_full(h1)
    _h2, _s1b, s2 = _tc_layer(agg1, h1, scale1, W1_1, b1_1r, W2_1, b2_1r)

    pw = jnp.stack([
        jnp.pad(pW0, ((0, 0), (0, d - o))),
        jnp.pad(pW1, ((0, 0), (0, d - o))),
        jnp.pad(pW2, ((0, 0), (0, d - o))),
    ])
    pb = jnp.pad(pb0 + pb1 + pb2, (0, d - o)).reshape(1, d)
    score = _score(sx, s1, s2, pw, pb)
    return score[0, :o]


# speedup vs baseline: 2.4853x; 2.4853x over previous
import functools

import jax
import jax.numpy as jnp
from jax import lax
from jax.experimental import pallas as pl
from jax.experimental.pallas import tpu as pltpu
from jax.experimental.pallas import tpu_sc as plsc

NC = 2
NS = 16
CHUNK = 128
NBUF = 8
NHALF = 2


def _sc_agg_body(npad, ch, hst, idx3, zer, out, idx_v, *rest):
    rows = rest[0:NBUF]
    sem_g = rest[NBUF:2 * NBUF]
    sem_s = rest[2 * NBUF:3 * NBUF]
    acc_sh = rest[3 * NBUF]
    c = lax.axis_index("c")
    s = lax.axis_index("s")
    rps = npad // NS
    ch2 = ch // NHALF
    groups = ch2 // NBUF
    h_c = hst.at[c]

    pltpu.sync_copy(zer.at[pl.ds(s * rps, rps)], acc_sh.at[pl.ds(s * rps, rps)])
    plsc.subcore_barrier()

    for half in range(NHALF):
        pltpu.sync_copy(idx3.at[pl.ds(s * ch + half * ch2, ch2)], idx_v)

        for b in range(NBUF):
            pltpu.async_copy(h_c.at[idx_v.at[b, 0]], rows[b], sem_g[b])

        def step(g, carry):
            base = g * NBUF
            descs = []
            for b in range(NBUF):
                j = base + b
                pltpu.make_async_copy(
                    h_c.at[idx_v.at[j, 0]], rows[b], sem_g[b]).wait()
                descs.append(pltpu.async_copy(
                    rows[b], acc_sh.at[idx_v.at[j, 1]], sem_s[b], add=True))
            for b in range(NBUF):
                descs[b].wait()

                @pl.when(g < groups - 1)
                def _():
                    pltpu.async_copy(
                        h_c.at[idx_v.at[base + NBUF + b, 0]], rows[b], sem_g[b])
            return carry

        lax.fori_loop(0, groups, step, 0)

    plsc.subcore_barrier()
    pltpu.sync_copy(acc_sh.at[pl.ds(s * rps, rps)],
                    out.at[c, pl.ds(s * rps, rps)])


def _sc_aggregate(hst, idx3, zer, npad, ch):
    hd = hst.shape[2]
    body = functools.partial(_sc_agg_body, npad, ch)
    return pl.kernel(
        body,
        out_type=jax.ShapeDtypeStruct((NC, npad, hd), jnp.float32),
        mesh=plsc.VectorSubcoreMesh(core_axis_name="c", subcore_axis_name="s"),
        compiler_params=pltpu.CompilerParams(use_tc_tiling_on_sc=False),
        scratch_types=[
            pltpu.VMEM((ch // NHALF, 2, CHUNK), jnp.int32),
            *[pltpu.VMEM((CHUNK, hd), jnp.float32) for _ in range(NBUF)],
            *[pltpu.SemaphoreType.DMA for _ in range(2 * NBUF)],
            pltpu.VMEM_SHARED((npad, hd), jnp.float32),
        ],
    )(hst, idx3, zer)


def _tc_layer_body(n, aggp, hst, scale, W1, b1, W2, b2, hout, sin, sout):
    hd = aggp.shape[2]
    agg = jnp.concatenate([aggp[0, :n, :], aggp[1, :n, :]], axis=1)
    hv = jnp.concatenate([hst[0], hst[1]], axis=1)
    u = agg + scale[0, 0] * hv
    a1 = jnp.maximum(
        jnp.dot(u, W1[...], preferred_element_type=jnp.float32) + b1[...], 0.0)
    u2 = jnp.dot(a1, W2[...], preferred_element_type=jnp.float32) + b2[...]
    m = jnp.mean(u2, axis=0, keepdims=True)
    var = jnp.mean(u2 * u2, axis=0, keepdims=True) - m * m
    hn = jnp.maximum((u2 - m) * lax.rsqrt(var + 1e-5), 0.0)
    hout[0, :, :] = hn[:, :hd]
    hout[1, :, :] = hn[:, hd:]
    sin[...] = jnp.sum(hv, axis=0, keepdims=True)
    sout[...] = jnp.sum(hn, axis=0, keepdims=True)


def _tc_layer(aggp, hst, scale, W1, b1, W2, b2):
    _, n, hd = hst.shape
    d = 2 * hd
    hdim = W1.shape[1]
    return pl.pallas_call(
        functools.partial(_tc_layer_body, n),
        out_shape=[
            jax.ShapeDtypeStruct((2, n, hdim // 2), jnp.float32),
            jax.ShapeDtypeStruct((1, d), jnp.float32),
            jax.ShapeDtypeStruct((1, hdim), jnp.float32),
        ],
        in_specs=[
            pl.BlockSpec(memory_space=pltpu.VMEM),
            pl.BlockSpec(memory_space=pltpu.VMEM),
            pl.BlockSpec(memory_space=pltpu.SMEM),
            pl.BlockSpec(memory_space=pltpu.VMEM),
            pl.BlockSpec(memory_space=pltpu.VMEM),
            pl.BlockSpec(memory_space=pltpu.VMEM),
            pl.BlockSpec(memory_space=pltpu.VMEM),
        ],
        out_specs=[
            pl.BlockSpec(memory_space=pltpu.VMEM),
            pl.BlockSpec(memory_space=pltpu.VMEM),
            pl.BlockSpec(memory_space=pltpu.VMEM),
        ],
    )(aggp, hst, scale, W1, b1, W2, b2)


def _score_body(sx, s1, s2, pw, pb, out):
    r = (jnp.dot(sx[...], pw[0], preferred_element_type=jnp.float32)
         + jnp.dot(s1[...], pw[1], preferred_element_type=jnp.float32)
         + jnp.dot(s2[...], pw[2], preferred_element_type=jnp.float32)
         + pb[...])
    out[...] = r


def _score(sx, s1, s2, pw, pb):
    d = pw.shape[2]
    return pl.pallas_call(
        _score_body,
        out_shape=jax.ShapeDtypeStruct((1, d), jnp.float32),
        in_specs=[pl.BlockSpec(memory_space=pltpu.VMEM)] * 5,
        out_specs=pl.BlockSpec(memory_space=pltpu.VMEM),
    )(sx, s1, s2, pw, pb)


def kernel(x, edge_index, eps, W1_0, b1_0, W2_0, b2_0, W1_1, b1_1, W2_1, b2_1,
           pW0, pb0, pW1, pb1, pW2, pb2):
    n, d = x.shape
    hd = d // 2
    e = edge_index.shape[1]
    o = pW0.shape[1]

    rows = -(-e // CHUNK)
    q = NS * max(8, NHALF * NBUF)
    rows_pad = -(-rows // q) * q
    ch = rows_pad // NS
    epad = rows_pad * CHUNK
    npad = -(-(n + 1) // (NS * 8)) * (NS * 8)

    srcr = jnp.concatenate(
        [edge_index[0], jnp.zeros((epad - e,), jnp.int32)]).reshape(rows_pad, CHUNK)
    dstr = jnp.concatenate(
        [edge_index[1], jnp.full((epad - e,), n, jnp.int32)]).reshape(rows_pad, CHUNK)
    idx3 = jnp.stack([srcr, dstr], axis=1)
    zer = jnp.zeros((npad, hd), jnp.float32)
    xst = x.reshape(n, 2, hd).swapaxes(0, 1)

    scale0 = (1.0 + eps[0]).reshape(1, 1)
    scale1 = (1.0 + eps[1]).reshape(1, 1)
    b1_0r, b2_0r = b1_0.reshape(1, -1), b2_0.reshape(1, -1)
    b1_1r, b2_1r = b1_1.reshape(1, -1), b2_1.reshape(1, -1)

    agg0 = _sc_aggregate(xst, idx3, zer, npad, ch)
    h1st, sx, s1 = _tc_layer(agg0, xst, scale0, W1_0, b1_0r, W2_0, b2_0r)
    agg1 = _sc_aggregate(h1st, idx3, zer, npad, ch)
    _h2st, _s1b, s2 = _tc_layer(agg1, h1st, scale1, W1_1, b1_1r, W2_1, b2_1r)

    pw = jnp.stack([
        jnp.pad(pW0, ((0, 0), (0, d - o))),
        jnp.pad(pW1, ((0, 0), (0, d - o))),
        jnp.pad(pW2, ((0, 0), (0, d - o))),
    ])
    pb = jnp.pad(pb0 + pb1 + pb2, (0, d - o)).reshape(1, d)
    score = _score(sx, s1, s2, pw, pb)
    return score[0, :o]


# speedup vs baseline: 2.5088x; 1.0094x over previous
import functools

import jax
import jax.numpy as jnp
from jax import lax
from jax.experimental import pallas as pl
from jax.experimental.pallas import tpu as pltpu
from jax.experimental.pallas import tpu_sc as plsc

NC = 2
NS = 16
CHUNK = 128
NBUF = 8
NHALF = 2


def _sc_agg_body(npad, ch, hst, idx3, zer, out, idx_v, *rest):
    rows = rest[0:NBUF]
    sem_g = rest[NBUF:2 * NBUF]
    sem_s = rest[2 * NBUF:3 * NBUF]
    acc_sh = rest[3 * NBUF]
    c = lax.axis_index("c")
    s = lax.axis_index("s")
    rps = npad // NS
    ch2 = ch // NHALF
    groups = ch2 // NBUF
    h_c = hst.at[c]

    pltpu.sync_copy(zer.at[pl.ds(s * rps, rps)], acc_sh.at[pl.ds(s * rps, rps)])
    plsc.subcore_barrier()

    for half in range(NHALF):
        pltpu.sync_copy(idx3.at[pl.ds(s * ch + half * ch2, ch2)], idx_v)

        for b in range(NBUF):
            pltpu.async_copy(h_c.at[idx_v.at[b, 0]], rows[b], sem_g[b])

        def step(g, carry):
            base = g * NBUF
            descs = []
            for b in range(NBUF):
                j = base + b
                pltpu.make_async_copy(
                    h_c.at[idx_v.at[j, 0]], rows[b], sem_g[b]).wait()
                descs.append(pltpu.async_copy(
                    rows[b], acc_sh.at[idx_v.at[j, 1]], sem_s[b], add=True))
            for b in range(NBUF):
                descs[b].wait()

                @pl.when(g < groups - 1)
                def _():
                    pltpu.async_copy(
                        h_c.at[idx_v.at[base + NBUF + b, 0]], rows[b], sem_g[b])
            return carry

        lax.fori_loop(0, groups, step, 0)

    plsc.subcore_barrier()
    pltpu.sync_copy(acc_sh.at[pl.ds(s * rps, rps)],
                    out.at[c, pl.ds(s * rps, rps)])


def _sc_aggregate(hst, idx3, zer, npad, ch):
    hd = hst.shape[2]
    body = functools.partial(_sc_agg_body, npad, ch)
    return pl.kernel(
        body,
        out_type=jax.ShapeDtypeStruct((NC, npad, hd), jnp.float32),
        mesh=plsc.VectorSubcoreMesh(core_axis_name="c", subcore_axis_name="s"),
        compiler_params=pltpu.CompilerParams(use_tc_tiling_on_sc=False),
        scratch_types=[
            pltpu.VMEM((ch // NHALF, 2, CHUNK), jnp.int32),
            *[pltpu.VMEM((CHUNK, hd), jnp.float32) for _ in range(NBUF)],
            *[pltpu.SemaphoreType.DMA for _ in range(2 * NBUF)],
            pltpu.VMEM_SHARED((npad, hd), jnp.float32),
        ],
    )(hst, idx3, zer)


def _tc_layer_body(n, aggp, hst, scale, W1, b1, W2, b2, hout, sin, sout):
    hd = aggp.shape[2]
    agg = jnp.concatenate([aggp[0, :n, :], aggp[1, :n, :]], axis=1)
    hv = jnp.concatenate([hst[0], hst[1]], axis=1)
    u = agg + scale[0, 0] * hv
    a1 = jnp.maximum(
        jnp.dot(u, W1[...], preferred_element_type=jnp.float32) + b1[...], 0.0)
    u2 = jnp.dot(a1, W2[...], preferred_element_type=jnp.float32) + b2[...]
    m = jnp.mean(u2, axis=0, keepdims=True)
    var = jnp.mean(u2 * u2, axis=0, keepdims=True) - m * m
    hn = jnp.maximum((u2 - m) * lax.rsqrt(var + 1e-5), 0.0)
    hout[0, :, :] = hn[:, :hd]
    hout[1, :, :] = hn[:, hd:]
    sin[...] = jnp.sum(hv, axis=0, keepdims=True)
    sout[...] = jnp.sum(hn, axis=0, keepdims=True)


def _tc_layer(aggp, hst, scale, W1, b1, W2, b2):
    _, n, hd = hst.shape
    d = 2 * hd
    hdim = W1.shape[1]
    return pl.pallas_call(
        functools.partial(_tc_layer_body, n),
        out_shape=[
            jax.ShapeDtypeStruct((2, n, hdim // 2), jnp.float32),
            jax.ShapeDtypeStruct((1, d), jnp.float32),
            jax.ShapeDtypeStruct((1, hdim), jnp.float32),
        ],
        in_specs=[
            pl.BlockSpec(memory_space=pltpu.VMEM),
            pl.BlockSpec(memory_space=pltpu.VMEM),
            pl.BlockSpec(memory_space=pltpu.SMEM),
            pl.BlockSpec(memory_space=pltpu.VMEM),
            pl.BlockSpec(memory_space=pltpu.VMEM),
            pl.BlockSpec(memory_space=pltpu.VMEM),
            pl.BlockSpec(memory_space=pltpu.VMEM),
        ],
        out_specs=[
            pl.BlockSpec(memory_space=pltpu.VMEM),
            pl.BlockSpec(memory_space=pltpu.VMEM),
            pl.BlockSpec(memory_space=pltpu.VMEM),
        ],
    )(aggp, hst, scale, W1, b1, W2, b2)


def _tc_layer2_body(n, aggp, hst, scale, W1, b1, W2, b2, sx, s1, pw, pb, out):
    agg = jnp.concatenate([aggp[0, :n, :], aggp[1, :n, :]], axis=1)
    hv = jnp.concatenate([hst[0], hst[1]], axis=1)
    u = agg + scale[0, 0] * hv
    a1 = jnp.maximum(
        jnp.dot(u, W1[...], preferred_element_type=jnp.float32) + b1[...], 0.0)
    u2 = jnp.dot(a1, W2[...], preferred_element_type=jnp.float32) + b2[...]
    m = jnp.mean(u2, axis=0, keepdims=True)
    var = jnp.mean(u2 * u2, axis=0, keepdims=True) - m * m
    hn = jnp.maximum((u2 - m) * lax.rsqrt(var + 1e-5), 0.0)
    s2 = jnp.sum(hn, axis=0, keepdims=True)
    out[...] = (jnp.dot(sx[...], pw[0], preferred_element_type=jnp.float32)
                + jnp.dot(s1[...], pw[1], preferred_element_type=jnp.float32)
                + jnp.dot(s2, pw[2], preferred_element_type=jnp.float32)
                + pb[...])


def _tc_layer2(aggp, hst, scale, W1, b1, W2, b2, sx, s1, pw, pb):
    n = hst.shape[1]
    d = pw.shape[2]
    return pl.pallas_call(
        functools.partial(_tc_layer2_body, n),
        out_shape=jax.ShapeDtypeStruct((1, d), jnp.float32),
        in_specs=[
            pl.BlockSpec(memory_space=pltpu.VMEM),
            pl.BlockSpec(memory_space=pltpu.VMEM),
            pl.BlockSpec(memory_space=pltpu.SMEM),
            pl.BlockSpec(memory_space=pltpu.VMEM),
            pl.BlockSpec(memory_space=pltpu.VMEM),
            pl.BlockSpec(memory_space=pltpu.VMEM),
            pl.BlockSpec(memory_space=pltpu.VMEM),
            pl.BlockSpec(memory_space=pltpu.VMEM),
            pl.BlockSpec(memory_space=pltpu.VMEM),
            pl.BlockSpec(memory_space=pltpu.VMEM),
            pl.BlockSpec(memory_space=pltpu.VMEM),
        ],
        out_specs=pl.BlockSpec(memory_space=pltpu.VMEM),
    )(aggp, hst, scale, W1, b1, W2, b2, sx, s1, pw, pb)


def kernel(x, edge_index, eps, W1_0, b1_0, W2_0, b2_0, W1_1, b1_1, W2_1, b2_1,
           pW0, pb0, pW1, pb1, pW2, pb2):
    n, d = x.shape
    hd = d // 2
    e = edge_index.shape[1]
    o = pW0.shape[1]

    rows = -(-e // CHUNK)
    q = NS * max(8, NHALF * NBUF)
    rows_pad = -(-rows // q) * q
    ch = rows_pad // NS
    epad = rows_pad * CHUNK
    npad = -(-(n + 1) // (NS * 8)) * (NS * 8)

    srcr = jnp.concatenate(
        [edge_index[0], jnp.zeros((epad - e,), jnp.int32)]).reshape(rows_pad, CHUNK)
    dstr = jnp.concatenate(
        [edge_index[1], jnp.full((epad - e,), n, jnp.int32)]).reshape(rows_pad, CHUNK)
    idx3 = jnp.stack([srcr, dstr], axis=1)
    zer = jnp.zeros((npad, hd), jnp.float32)
    xst = x.reshape(n, 2, hd).swapaxes(0, 1)

    scale0 = (1.0 + eps[0]).reshape(1, 1)
    scale1 = (1.0 + eps[1]).reshape(1, 1)
    b1_0r, b2_0r = b1_0.reshape(1, -1), b2_0.reshape(1, -1)
    b1_1r, b2_1r = b1_1.reshape(1, -1), b2_1.reshape(1, -1)

    pw = jnp.stack([
        jnp.pad(pW0, ((0, 0), (0, d - o))),
        jnp.pad(pW1, ((0, 0), (0, d - o))),
        jnp.pad(pW2, ((0, 0), (0, d - o))),
    ])
    pb = jnp.pad(pb0 + pb1 + pb2, (0, d - o)).reshape(1, d)

    agg0 = _sc_aggregate(xst, idx3, zer, npad, ch)
    h1st, sx, s1 = _tc_layer(agg0, xst, scale0, W1_0, b1_0r, W2_0, b2_0r)
    agg1 = _sc_aggregate(h1st, idx3, zer, npad, ch)
    score = _tc_layer2(agg1, h1st, scale1, W1_1, b1_1r, W2_1, b2_1r,
                       sx, s1, pw, pb)
    return score[0, :o]
